# baseline (speedup 1.0000x reference)
import functools

import jax
import jax.numpy as jnp
from jax import lax
from jax.experimental import pallas as pl
from jax.experimental.pallas import tpu as pltpu
from jax.experimental.pallas import tpu_sc as plsc

_B = 4096
_L = 20
_D = 16
_C = 3 * _D
_ITEM_NUM = 1000000
_ATTR_NUM = 100000
_BL = _B * _L
_NW = 32

_CH = 1536
_I_COLS = (_ITEM_NUM // _CH) * _CH
_A_COLS = (_ATTR_NUM // _CH) * _CH
_I_NCH = _I_COLS // _CH
_A_NCH = _A_COLS // _CH
_NCH = _I_NCH + _A_NCH
_DR = _CH // 8

_B_W = _B // _NW
_PASS_B = 32
_NPASS = _B_W // _PASS_B
_RP = _PASS_B * _L
_GROUPS = _RP // 16


@functools.partial(
    pl.kernel,
    mesh=plsc.VectorSubcoreMesh(core_axis_name="c", subcore_axis_name="s"),
    out_type=(jax.ShapeDtypeStruct((_ITEM_NUM // 8, 128), jnp.float32),
              jax.ShapeDtypeStruct((_ATTR_NUM // 8 + 4, 128), jnp.float32)),
    compiler_params=pltpu.CompilerParams(needs_layout_passes=False),
    scratch_types=[
        pltpu.VMEM((16, _CH), jnp.float32),
        pltpu.VMEM((_DR, 128), jnp.float32),
        pltpu.SemaphoreType.DMA,
    ],
)
def _transpose_tabs(t2i_hbm, t2a_hbm, taili_hbm, taila_hbm,
                    di_hbm, da_hbm, blk, dv, sem):
    wid = lax.axis_index("s") * 2 + lax.axis_index("c")

    def retile(src_hbm, dst_hbm, ch_local):
        c0 = pl.multiple_of(ch_local * _CH, _CH)
        pltpu.sync_copy(src_hbm.at[:, pl.ds(c0, _CH)], blk)

        def row_body(r, _):
            for kk in range(8):
                col = plsc.load_gather(
                    blk, [lax.iota(jnp.int32, 16),
                          jnp.full((16,), r * 8 + kk, jnp.int32)])
                plsc.store_scatter(
                    dv, [jnp.full((16,), r, jnp.int32),
                         lax.iota(jnp.int32, 16) + kk * 16], col)
            return 0

        lax.fori_loop(0, _DR, row_body, 0)
        pltpu.sync_copy(
            dv, dst_hbm.at[pl.ds(pl.multiple_of(c0 // 8, _DR), _DR)])

    def chunk_body(k, _):
        q = k * _NW + wid

        @pl.when(q < _I_NCH)
        def _():
            retile(t2i_hbm, di_hbm, q)

        @pl.when((q >= _I_NCH) & (q < _NCH))
        def _():
            retile(t2a_hbm, da_hbm, q - _I_NCH)

        return 0

    lax.fori_loop(0, (_NCH + _NW - 1) // _NW, chunk_body, 0)

    @pl.when(wid == 0)
    def _():
        pltpu.sync_copy(taili_hbm, dv.at[pl.ds(0, 8)])
        pltpu.sync_copy(dv.at[pl.ds(0, 8)], di_hbm.at[pl.ds(_I_COLS // 8, 8)])

    @pl.when(wid == 1)
    def _():
        pltpu.sync_copy(taila_hbm, dv.at[pl.ds(8, 24)])
        pltpu.sync_copy(dv.at[pl.ds(8, 24)], da_hbm.at[pl.ds(_A_COLS // 8, 24)])


@functools.partial(
    pl.kernel,
    mesh=plsc.VectorSubcoreMesh(core_axis_name="c", subcore_axis_name="s"),
    out_type=jax.ShapeDtypeStruct((_L * _C, _B), jnp.float32),
    compiler_params=pltpu.CompilerParams(
        use_tc_tiling_on_sc=False, needs_layout_passes=False),
    scratch_types=[
        pltpu.VMEM((_RP,), jnp.int32),
        pltpu.VMEM((_RP,), jnp.int32),
        pltpu.VMEM((_RP,), jnp.int32),
        pltpu.VMEM((_RP,), jnp.int32),
        pltpu.VMEM((_RP,), jnp.int32),
        pltpu.VMEM((_RP,), jnp.int32),
        pltpu.VMEM((_RP, 8 * _D), jnp.float32),
        pltpu.VMEM((_L * _C, _PASS_B), jnp.float32),
        pltpu.SemaphoreType.DMA,
    ],
)
def _gather_concat(iid_hbm, a0_hbm, a1_hbm, lrow_hbm, bcol_hbm,
                   item_t, attr_t, out_hbm,
                   ii_v, i0_v, i1_v, tt_v, lr_v, bc_v, tiles_v, out_v, sem):
    wid = lax.axis_index("s") * 2 + lax.axis_index("c")

    def pass_body(p, _):
        b0 = wid * _B_W + p * _PASS_B
        i0 = b0 * _L
        pltpu.sync_copy(iid_hbm.at[pl.ds(i0, _RP)], ii_v)
        pltpu.sync_copy(a0_hbm.at[pl.ds(i0, _RP)], i0_v)
        pltpu.sync_copy(a1_hbm.at[pl.ds(i0, _RP)], i1_v)
        pltpu.sync_copy(lrow_hbm.at[pl.ds(i0, _RP)], lr_v)
        pltpu.sync_copy(bcol_hbm.at[pl.ds(i0, _RP)], bc_v)

        for idx_v, table, c0 in ((ii_v, item_t, 0),
                                 (i0_v, attr_t, _D),
                                 (i1_v, attr_t, 2 * _D)):
            def tile_ids(g, _):
                tt_v[pl.ds(g * 16, 16)] = idx_v[pl.ds(g * 16, 16)] >> 3
                return 0

            lax.fori_loop(0, _GROUPS, tile_ids, 0)
            pltpu.async_copy(table.at[tt_v], tiles_v, sem).wait()

            def group_body(g, _):
                j16 = lax.iota(jnp.int32, 16) + g * 16
                cb16 = (idx_v[pl.ds(g * 16, 16)] & 7) << 4
                lr16 = lr_v[pl.ds(g * 16, 16)] + c0
                bc16 = bc_v[pl.ds(g * 16, 16)] - b0
                for d in range(_D):
                    v = plsc.load_gather(tiles_v, [j16, cb16 + d])
                    plsc.store_scatter(out_v, [lr16 + d, bc16], v)
                return 0

            lax.fori_loop(0, _GROUPS, group_body, 0)

        pltpu.sync_copy(out_v, out_hbm.at[:, pl.ds(b0, _PASS_B)])
        return 0

    lax.fori_loop(0, _NPASS, pass_body, 0)


def kernel(hist_iid_seq, hist_aid_seq, hist_rate_seq, hist_seq_len,
           iid_list, aid_list, lb_list,
           item_table, attr_table, rating_table):
    tail_i = item_table[_I_COLS:_ITEM_NUM].reshape(8, 128)
    tail_a = jnp.pad(
        attr_table[_A_COLS:_ATTR_NUM].reshape(20, 128), ((0, 4), (0, 0)))
    item2, attr2 = _transpose_tabs(
        item_table.T, attr_table.T, tail_i, tail_a)
    iid = iid_list.reshape(_BL).astype(jnp.int32)
    a0 = aid_list[:, :, 0].reshape(_BL).astype(jnp.int32)
    a1 = aid_list[:, :, 1].reshape(_BL).astype(jnp.int32)
    ar = jnp.arange(_BL, dtype=jnp.int32)
    lrow = (ar % _L) * _C
    bcol = ar // _L
    out4 = _gather_concat(iid, a0, a1, lrow, bcol, item2, attr2)
    out = out4.reshape(_L, _C, _B).transpose(2, 0, 1)
    return out, lb_list.reshape(_B, _L)


# speedup vs baseline: 1.7441x; 1.7441x over previous
import functools

import jax
import jax.numpy as jnp
from jax import lax
from jax.experimental import pallas as pl
from jax.experimental.pallas import tpu as pltpu
from jax.experimental.pallas import tpu_sc as plsc

_B = 4096
_L = 20
_D = 16
_C = 3 * _D
_ITEM_NUM = 1000000
_ATTR_NUM = 100000
_BL = _B * _L
_NW = 32

_CH = 1536
_I_COLS = (_ITEM_NUM // _CH) * _CH
_A_COLS = (_ATTR_NUM // _CH) * _CH
_I_NCH = _I_COLS // _CH
_A_NCH = _A_COLS // _CH
_NCH = _I_NCH + _A_NCH
_DR = _CH // 8

_B_W = _B // _NW
_PASS_B = 32
_NPASS = _B_W // _PASS_B
_RP = _PASS_B * _L
_GROUPS = _RP // 16


@functools.partial(
    pl.kernel,
    mesh=plsc.VectorSubcoreMesh(core_axis_name="c", subcore_axis_name="s"),
    out_type=(jax.ShapeDtypeStruct((_ITEM_NUM // 8, 128), jnp.float32),
              jax.ShapeDtypeStruct((_ATTR_NUM // 8 + 4, 128), jnp.float32)),
    compiler_params=pltpu.CompilerParams(needs_layout_passes=False),
    scratch_types=[
        pltpu.VMEM((16, _CH), jnp.float32),
        pltpu.VMEM((_DR, 128), jnp.float32),
        pltpu.SemaphoreType.DMA,
    ],
)
def _transpose_tabs(t2i_hbm, t2a_hbm, taili_hbm, taila_hbm,
                    di_hbm, da_hbm, blk, dv, sem):
    wid = lax.axis_index("s") * 2 + lax.axis_index("c")

    def retile(src_hbm, dst_hbm, ch_local):
        c0 = pl.multiple_of(ch_local * _CH, _CH)
        pltpu.sync_copy(src_hbm.at[:, pl.ds(c0, _CH)], blk)
        iota = lax.iota(jnp.int32, 16)
        rhalf = iota >> 3
        cbase = (iota & 7) << 4

        def seg_body(m, _):
            r16 = rhalf + m * 2
            for d in range(16):
                seg = blk[d, pl.ds(m * 16, 16)]
                plsc.store_scatter(dv, [r16, cbase + d], seg)
            return 0

        lax.fori_loop(0, _CH // 16, seg_body, 0)
        pltpu.sync_copy(
            dv, dst_hbm.at[pl.ds(pl.multiple_of(c0 // 8, _DR), _DR)])

    def chunk_body(k, _):
        q = k * _NW + wid

        @pl.when(q < _I_NCH)
        def _():
            retile(t2i_hbm, di_hbm, q)

        @pl.when((q >= _I_NCH) & (q < _NCH))
        def _():
            retile(t2a_hbm, da_hbm, q - _I_NCH)

        return 0

    lax.fori_loop(0, (_NCH + _NW - 1) // _NW, chunk_body, 0)

    @pl.when(wid == 0)
    def _():
        pltpu.sync_copy(taili_hbm, dv.at[pl.ds(0, 8)])
        pltpu.sync_copy(dv.at[pl.ds(0, 8)], di_hbm.at[pl.ds(_I_COLS // 8, 8)])

    @pl.when(wid == 1)
    def _():
        pltpu.sync_copy(taila_hbm, dv.at[pl.ds(8, 24)])
        pltpu.sync_copy(dv.at[pl.ds(8, 24)], da_hbm.at[pl.ds(_A_COLS // 8, 24)])


@functools.partial(
    pl.kernel,
    mesh=plsc.VectorSubcoreMesh(core_axis_name="c", subcore_axis_name="s"),
    out_type=jax.ShapeDtypeStruct((_L * _C, _B), jnp.float32),
    compiler_params=pltpu.CompilerParams(
        use_tc_tiling_on_sc=False, needs_layout_passes=False),
    scratch_types=[
        pltpu.VMEM((_RP,), jnp.int32),
        pltpu.VMEM((_RP,), jnp.int32),
        pltpu.VMEM((_RP,), jnp.int32),
        pltpu.VMEM((_RP,), jnp.int32),
        pltpu.VMEM((_RP,), jnp.int32),
        pltpu.VMEM((_RP,), jnp.int32),
        pltpu.VMEM((_RP, 8 * _D), jnp.float32),
        pltpu.VMEM((_L * _C, _PASS_B), jnp.float32),
        pltpu.SemaphoreType.DMA,
    ],
)
def _gather_concat(iid_hbm, a0_hbm, a1_hbm, lrow_hbm, bcol_hbm,
                   item_t, attr_t, out_hbm,
                   ii_v, i0_v, i1_v, tt_v, lr_v, bc_v, tiles_v, out_v, sem):
    wid = lax.axis_index("s") * 2 + lax.axis_index("c")

    def pass_body(p, _):
        b0 = wid * _B_W + p * _PASS_B
        i0 = b0 * _L
        pltpu.sync_copy(iid_hbm.at[pl.ds(i0, _RP)], ii_v)
        pltpu.sync_copy(a0_hbm.at[pl.ds(i0, _RP)], i0_v)
        pltpu.sync_copy(a1_hbm.at[pl.ds(i0, _RP)], i1_v)
        pltpu.sync_copy(lrow_hbm.at[pl.ds(i0, _RP)], lr_v)
        pltpu.sync_copy(bcol_hbm.at[pl.ds(i0, _RP)], bc_v)

        for idx_v, table, c0 in ((ii_v, item_t, 0),
                                 (i0_v, attr_t, _D),
                                 (i1_v, attr_t, 2 * _D)):
            def tile_ids(g, _):
                tt_v[pl.ds(g * 16, 16)] = idx_v[pl.ds(g * 16, 16)] >> 3
                return 0

            lax.fori_loop(0, _GROUPS, tile_ids, 0)
            pltpu.async_copy(table.at[tt_v], tiles_v, sem).wait()

            def group_body(g, _):
                j16 = lax.iota(jnp.int32, 16) + g * 16
                cb16 = (idx_v[pl.ds(g * 16, 16)] & 7) << 4
                lr16 = lr_v[pl.ds(g * 16, 16)] + c0
                bc16 = bc_v[pl.ds(g * 16, 16)] - b0
                for d in range(_D):
                    v = plsc.load_gather(tiles_v, [j16, cb16 + d])
                    plsc.store_scatter(out_v, [lr16 + d, bc16], v)
                return 0

            lax.fori_loop(0, _GROUPS, group_body, 0)

        pltpu.sync_copy(out_v, out_hbm.at[:, pl.ds(b0, _PASS_B)])
        return 0

    lax.fori_loop(0, _NPASS, pass_body, 0)


def kernel(hist_iid_seq, hist_aid_seq, hist_rate_seq, hist_seq_len,
           iid_list, aid_list, lb_list,
           item_table, attr_table, rating_table):
    tail_i = item_table[_I_COLS:_ITEM_NUM].reshape(8, 128)
    tail_a = jnp.pad(
        attr_table[_A_COLS:_ATTR_NUM].reshape(20, 128), ((0, 4), (0, 0)))
    item2, attr2 = _transpose_tabs(
        item_table.T, attr_table.T, tail_i, tail_a)
    iid = iid_list.reshape(_BL).astype(jnp.int32)
    a0 = aid_list[:, :, 0].reshape(_BL).astype(jnp.int32)
    a1 = aid_list[:, :, 1].reshape(_BL).astype(jnp.int32)
    ar = jnp.arange(_BL, dtype=jnp.int32)
    lrow = (ar % _L) * _C
    bcol = ar // _L
    out4 = _gather_concat(iid, a0, a1, lrow, bcol, item2, attr2)
    out = out4.reshape(_L, _C, _B).transpose(2, 0, 1)
    return out, lb_list.reshape(_B, _L)


# speedup vs baseline: 1.7726x; 1.0163x over previous
import functools

import jax
import jax.numpy as jnp
from jax import lax
from jax.experimental import pallas as pl
from jax.experimental.pallas import tpu as pltpu
from jax.experimental.pallas import tpu_sc as plsc

_B = 4096
_L = 20
_D = 16
_C = 3 * _D
_ITEM_NUM = 1000000
_ATTR_NUM = 100000
_BL = _B * _L
_NW = 32

_CH = 1536
_I_COLS = (_ITEM_NUM // _CH) * _CH
_A_COLS = (_ATTR_NUM // _CH) * _CH
_I_NCH = _I_COLS // _CH
_A_NCH = _A_COLS // _CH
_NCH = _I_NCH + _A_NCH
_DR = _CH // 8

_B_W = _B // _NW
_PASS_B = 32
_NPASS = _B_W // _PASS_B
_RP = _PASS_B * _L
_GROUPS = _RP // 16


@functools.partial(
    pl.kernel,
    mesh=plsc.VectorSubcoreMesh(core_axis_name="c", subcore_axis_name="s"),
    out_type=(jax.ShapeDtypeStruct((_ITEM_NUM // 8, 128), jnp.float32),
              jax.ShapeDtypeStruct((_ATTR_NUM // 8 + 4, 128), jnp.float32)),
    compiler_params=pltpu.CompilerParams(needs_layout_passes=False),
    scratch_types=[
        pltpu.VMEM((16, _CH), jnp.float32),
        pltpu.VMEM((_DR, 128), jnp.float32),
        pltpu.SemaphoreType.DMA,
    ],
)
def _transpose_tabs(t2i_hbm, t2a_hbm, taili_hbm, taila_hbm,
                    di_hbm, da_hbm, blk, dv, sem):
    wid = lax.axis_index("s") * 2 + lax.axis_index("c")

    def retile(src_hbm, dst_hbm, ch_local):
        c0 = pl.multiple_of(ch_local * _CH, _CH)
        pltpu.sync_copy(src_hbm.at[:, pl.ds(c0, _CH)], blk)
        iota = lax.iota(jnp.int32, 16)
        rhalf = iota >> 3
        cbase = (iota & 7) << 4

        def seg_body(m, _):
            r16 = rhalf + m * 2
            for d in range(16):
                seg = blk[d, pl.ds(m * 16, 16)]
                plsc.store_scatter(dv, [r16, cbase + d], seg)
            return 0

        lax.fori_loop(0, _CH // 16, seg_body, 0)
        pltpu.sync_copy(
            dv, dst_hbm.at[pl.ds(pl.multiple_of(c0 // 8, _DR), _DR)])

    def chunk_body(k, _):
        q = k * _NW + wid

        @pl.when(q < _I_NCH)
        def _():
            retile(t2i_hbm, di_hbm, q)

        @pl.when((q >= _I_NCH) & (q < _NCH))
        def _():
            retile(t2a_hbm, da_hbm, q - _I_NCH)

        return 0

    lax.fori_loop(0, (_NCH + _NW - 1) // _NW, chunk_body, 0)

    @pl.when(wid == 0)
    def _():
        pltpu.sync_copy(taili_hbm, dv.at[pl.ds(0, 8)])
        pltpu.sync_copy(dv.at[pl.ds(0, 8)], di_hbm.at[pl.ds(_I_COLS // 8, 8)])

    @pl.when(wid == 1)
    def _():
        pltpu.sync_copy(taila_hbm, dv.at[pl.ds(8, 24)])
        pltpu.sync_copy(dv.at[pl.ds(8, 24)], da_hbm.at[pl.ds(_A_COLS // 8, 24)])


@functools.partial(
    pl.kernel,
    mesh=plsc.VectorSubcoreMesh(core_axis_name="c", subcore_axis_name="s"),
    out_type=jax.ShapeDtypeStruct((_L * _C, _B), jnp.float32),
    compiler_params=pltpu.CompilerParams(
        use_tc_tiling_on_sc=False, needs_layout_passes=False),
    scratch_types=[
        pltpu.VMEM((5, _RP), jnp.int32),
        pltpu.VMEM((_RP,), jnp.int32),
        pltpu.VMEM((_RP, 8 * _D), jnp.float32),
        pltpu.VMEM((_L * _C, _PASS_B), jnp.float32),
        pltpu.SemaphoreType.DMA,
    ],
)
def _gather_concat(idx5_hbm, item_t, attr_t, out_hbm,
                   idx5_v, tt_v, tiles_v, out_v, sem):
    wid = lax.axis_index("s") * 2 + lax.axis_index("c")

    def pass_body(p, _):
        b0 = wid * _B_W + p * _PASS_B
        i0 = b0 * _L
        pltpu.sync_copy(idx5_hbm.at[:, pl.ds(i0, _RP)], idx5_v)

        for st, (table, c0) in enumerate(((item_t, 0),
                                          (attr_t, _D),
                                          (attr_t, 2 * _D))):
            def tile_ids(g, _):
                tt_v[pl.ds(g * 16, 16)] = idx5_v[st, pl.ds(g * 16, 16)] >> 3
                return 0

            lax.fori_loop(0, _GROUPS, tile_ids, 0)
            pltpu.async_copy(table.at[tt_v], tiles_v, sem).wait()

            def group_body(g, _):
                j16 = lax.iota(jnp.int32, 16) + g * 16
                cb16 = (idx5_v[st, pl.ds(g * 16, 16)] & 7) << 4
                lr16 = idx5_v[3, pl.ds(g * 16, 16)] + c0
                bc16 = idx5_v[4, pl.ds(g * 16, 16)] - b0
                for d in range(_D):
                    v = plsc.load_gather(tiles_v, [j16, cb16 + d])
                    plsc.store_scatter(out_v, [lr16 + d, bc16], v)
                return 0

            lax.fori_loop(0, _GROUPS, group_body, 0)

        pltpu.sync_copy(out_v, out_hbm.at[:, pl.ds(b0, _PASS_B)])
        return 0

    lax.fori_loop(0, _NPASS, pass_body, 0)


def kernel(hist_iid_seq, hist_aid_seq, hist_rate_seq, hist_seq_len,
           iid_list, aid_list, lb_list,
           item_table, attr_table, rating_table):
    tail_i = item_table[_I_COLS:_ITEM_NUM].reshape(8, 128)
    tail_a = jnp.pad(
        attr_table[_A_COLS:_ATTR_NUM].reshape(20, 128), ((0, 4), (0, 0)))
    item2, attr2 = _transpose_tabs(
        item_table.T, attr_table.T, tail_i, tail_a)
    iid = iid_list.reshape(_BL).astype(jnp.int32)
    a0 = aid_list[:, :, 0].reshape(_BL).astype(jnp.int32)
    a1 = aid_list[:, :, 1].reshape(_BL).astype(jnp.int32)
    ar = jnp.arange(_BL, dtype=jnp.int32)
    lrow = (ar % _L) * _C
    bcol = ar // _L
    idx5 = jnp.stack([iid, a0, a1, lrow, bcol])
    out4 = _gather_concat(idx5, item2, attr2)
    out = out4.reshape(_L, _C, _B).transpose(2, 0, 1)
    return out, lb_list.reshape(_B, _L)


# speedup vs baseline: 1.8063x; 1.0190x over previous
import functools

import jax
import jax.numpy as jnp
from jax import lax
from jax.experimental import pallas as pl
from jax.experimental.pallas import tpu as pltpu
from jax.experimental.pallas import tpu_sc as plsc

_B = 4096
_L = 20
_D = 16
_C = 3 * _D
_ITEM_NUM = 1000000
_ATTR_NUM = 100000
_BL = _B * _L
_NW = 32

_CH = 2048
_I_COLS = (_ITEM_NUM // _CH) * _CH
_A_COLS = (_ATTR_NUM // _CH) * _CH
_I_NCH = _I_COLS // _CH
_A_NCH = _A_COLS // _CH
_NCH = _I_NCH + _A_NCH
_DR = _CH // 8
_I_TAIL = (_ITEM_NUM - _I_COLS) // 8
_A_TAIL = -(-((_ATTR_NUM - _A_COLS) // 8) // 8) * 8

_B_W = _B // _NW
_PASS_B = 32
_NPASS = _B_W // _PASS_B
_RP = _PASS_B * _L
_GROUPS = _RP // 16


@functools.partial(
    pl.kernel,
    mesh=plsc.VectorSubcoreMesh(core_axis_name="c", subcore_axis_name="s"),
    out_type=(jax.ShapeDtypeStruct((_ITEM_NUM // 8, 128), jnp.float32),
              jax.ShapeDtypeStruct((_A_COLS // 8 + _A_TAIL, 128),
                                   jnp.float32)),
    compiler_params=pltpu.CompilerParams(needs_layout_passes=False),
    scratch_types=[
        pltpu.VMEM((16, _CH), jnp.float32),
        pltpu.VMEM((_DR, 128), jnp.float32),
        pltpu.SemaphoreType.DMA,
    ],
)
def _transpose_tabs(t2i_hbm, t2a_hbm, taili_hbm, taila_hbm,
                    di_hbm, da_hbm, blk, dv, sem):
    wid = lax.axis_index("s") * 2 + lax.axis_index("c")

    def retile(src_hbm, dst_hbm, ch_local):
        c0 = pl.multiple_of(ch_local * _CH, _CH)
        pltpu.sync_copy(src_hbm.at[:, pl.ds(c0, _CH)], blk)
        iota = lax.iota(jnp.int32, 16)
        rhalf = iota >> 3
        cbase = (iota & 7) << 4

        def seg_body(m, _):
            r16 = rhalf + m * 2
            for d in range(16):
                seg = blk[d, pl.ds(m * 16, 16)]
                plsc.store_scatter(dv, [r16, cbase + d], seg)
            return 0

        lax.fori_loop(0, _CH // 16, seg_body, 0)
        pltpu.sync_copy(
            dv, dst_hbm.at[pl.ds(pl.multiple_of(c0 // 8, _DR), _DR)])

    def chunk_body(k, _):
        q = k * _NW + wid

        @pl.when(q < _I_NCH)
        def _():
            retile(t2i_hbm, di_hbm, q)

        @pl.when((q >= _I_NCH) & (q < _NCH))
        def _():
            retile(t2a_hbm, da_hbm, q - _I_NCH)

        return 0

    lax.fori_loop(0, (_NCH + _NW - 1) // _NW, chunk_body, 0)

    @pl.when(wid == 0)
    def _():
        pltpu.sync_copy(taili_hbm, dv.at[pl.ds(0, _I_TAIL)])
        pltpu.sync_copy(dv.at[pl.ds(0, _I_TAIL)],
                        di_hbm.at[pl.ds(_I_COLS // 8, _I_TAIL)])

    @pl.when(wid == 1)
    def _():
        pltpu.sync_copy(taila_hbm, dv.at[pl.ds(0, _A_TAIL)])
        pltpu.sync_copy(dv.at[pl.ds(0, _A_TAIL)],
                        da_hbm.at[pl.ds(_A_COLS // 8, _A_TAIL)])


@functools.partial(
    pl.kernel,
    mesh=plsc.VectorSubcoreMesh(core_axis_name="c", subcore_axis_name="s"),
    out_type=jax.ShapeDtypeStruct((_L * _C, _B), jnp.float32),
    compiler_params=pltpu.CompilerParams(
        use_tc_tiling_on_sc=False, needs_layout_passes=False),
    scratch_types=[
        pltpu.VMEM((5, _RP), jnp.int32),
        pltpu.VMEM((_RP,), jnp.int32),
        pltpu.VMEM((_RP, 8 * _D), jnp.float32),
        pltpu.VMEM((_L * _C, _PASS_B), jnp.float32),
        pltpu.SemaphoreType.DMA,
    ],
)
def _gather_concat(idx5_hbm, item_t, attr_t, out_hbm,
                   idx5_v, tt_v, tiles_v, out_v, sem):
    wid = lax.axis_index("s") * 2 + lax.axis_index("c")

    def pass_body(p, _):
        b0 = wid * _B_W + p * _PASS_B
        i0 = b0 * _L
        pltpu.sync_copy(idx5_hbm.at[:, pl.ds(i0, _RP)], idx5_v)

        for st, (table, c0) in enumerate(((item_t, 0),
                                          (attr_t, _D),
                                          (attr_t, 2 * _D))):
            def tile_ids(g, _):
                tt_v[pl.ds(g * 16, 16)] = idx5_v[st, pl.ds(g * 16, 16)] >> 3
                return 0

            lax.fori_loop(0, _GROUPS, tile_ids, 0)
            pltpu.async_copy(table.at[tt_v], tiles_v, sem).wait()

            def group_body(g, _):
                j16 = lax.iota(jnp.int32, 16) + g * 16
                cb16 = (idx5_v[st, pl.ds(g * 16, 16)] & 7) << 4
                lr16 = idx5_v[3, pl.ds(g * 16, 16)] + c0
                bc16 = idx5_v[4, pl.ds(g * 16, 16)] - b0
                for d in range(_D):
                    v = plsc.load_gather(tiles_v, [j16, cb16 + d])
                    plsc.store_scatter(out_v, [lr16 + d, bc16], v)
                return 0

            lax.fori_loop(0, _GROUPS, group_body, 0)

        pltpu.sync_copy(out_v, out_hbm.at[:, pl.ds(b0, _PASS_B)])
        return 0

    lax.fori_loop(0, _NPASS, pass_body, 0)


def kernel(hist_iid_seq, hist_aid_seq, hist_rate_seq, hist_seq_len,
           iid_list, aid_list, lb_list,
           item_table, attr_table, rating_table):
    tail_i = item_table[_I_COLS:_ITEM_NUM].reshape(_I_TAIL, 128)
    n_a = (_ATTR_NUM - _A_COLS) // 8
    tail_a = jnp.pad(
        attr_table[_A_COLS:_ATTR_NUM].reshape(n_a, 128),
        ((0, _A_TAIL - n_a), (0, 0)))
    item2, attr2 = _transpose_tabs(
        item_table.T, attr_table.T, tail_i, tail_a)
    iid = iid_list.reshape(_BL).astype(jnp.int32)
    a0 = aid_list[:, :, 0].reshape(_BL).astype(jnp.int32)
    a1 = aid_list[:, :, 1].reshape(_BL).astype(jnp.int32)
    ar = jnp.arange(_BL, dtype=jnp.int32)
    lrow = (ar % _L) * _C
    bcol = ar // _L
    idx5 = jnp.stack([iid, a0, a1, lrow, bcol])
    out4 = _gather_concat(idx5, item2, attr2)
    out = out4.reshape(_L, _C, _B).transpose(2, 0, 1)
    return out, lb_list.reshape(_B, _L)
